# baseline (speedup 1.0000x reference)
import jax
import jax.numpy as jnp
from jax.experimental import pallas as pl

_B, _N, _F = 1024, 20, 19
_V = 101
_C = 128
_BB = 64


def _diag_embed_kernel(idx_ref, embT_ref, out_ref):
    flat = idx_ref[0]
    rows = jax.lax.broadcasted_iota(jnp.int32, (_V, _BB * _N), 0)
    onehot = (rows == flat).astype(jnp.float32)
    gT = jnp.dot(embT_ref[...], onehot,
                 preferred_element_type=jnp.float32)
    n_iota = jax.lax.broadcasted_iota(jnp.int32, (_N, _N * _N), 0)
    j_iota = jax.lax.broadcasted_iota(jnp.int32, (_N, _N * _N), 1)
    sel = (j_iota == (_N + 1) * n_iota).astype(jnp.float32)
    for b in range(_BB):
        t = gT[:, b * _N:(b + 1) * _N]
        out_ref[b] = jnp.dot(t, sel,
                             preferred_element_type=jnp.float32)


def kernel(batch_node_attr, emb_table):
    idx = batch_node_attr[:, :, 0].astype(jnp.int32)
    idx = idx.reshape(_B // _BB, 1, _BB * _N)
    embT = emb_table.T
    out = pl.pallas_call(
        _diag_embed_kernel,
        grid=(_B // _BB,),
        in_specs=[
            pl.BlockSpec((1, 1, _BB * _N), lambda i: (i, 0, 0)),
            pl.BlockSpec((_C, _V), lambda i: (0, 0)),
        ],
        out_specs=pl.BlockSpec((_BB, _C, _N * _N), lambda i: (i, 0, 0)),
        out_shape=jax.ShapeDtypeStruct((_B, _C, _N * _N), jnp.float32),
    )(idx, embT)
    return out.reshape(_B, _C, _N, _N)
